# baseline (speedup 1.0000x reference)
import functools

import jax
import jax.numpy as jnp
from jax import lax
from jax.experimental import pallas as pl
from jax.experimental.pallas import tpu as pltpu
from jax.experimental.pallas import tpu_sc as plsc

N = 10000
D_IN = 128
NCLS = 64
KL = 10
ALPHA = 0.1

NC = 2
NS = 16
NW = NC * NS
C = 128
NPAD = 10240
RPT = NPAD // NS

NBUF = 6
LAG = 2


def _make_edge_pass(num_chunks: int):
    assert num_chunks > NBUF
    mesh = plsc.VectorSubcoreMesh(core_axis_name="c", subcore_axis_name="s")

    @functools.partial(
        pl.kernel,
        out_type=jax.ShapeDtypeStruct((NC, NPAD, NCLS), jnp.float32),
        mesh=mesh,
        compiler_params=pltpu.CompilerParams(use_tc_tiling_on_sc=False),
        scratch_types=[
            pltpu.VMEM((num_chunks, C), jnp.int32),
            pltpu.VMEM((num_chunks, C), jnp.int32),
            [pltpu.VMEM((C, NCLS), jnp.float32)] * NBUF,
            pltpu.VMEM_SHARED((NPAD, NCLS), jnp.float32),
            [pltpu.SemaphoreType.DMA] * 3,
            [pltpu.SemaphoreType.DMA] * NBUF,
            [pltpu.SemaphoreType.DMA] * NBUF,
        ],
    )
    def edge_pass(z_hbm, src_hbm, dst_hbm, zeros_hbm, out_hbm,
                  src_all, dst_all, rows, acc, sems, gsems, ssems):
        cid = lax.axis_index("c")
        sid = lax.axis_index("s")
        wid = sid * NC + cid

        zd = pltpu.async_copy(zeros_hbm, acc.at[pl.ds(sid * RPT, RPT)],
                              sems[0])
        sd0 = pltpu.async_copy(src_hbm.at[wid], src_all, sems[1])
        sd1 = pltpu.async_copy(dst_hbm.at[wid], dst_all, sems[2])
        zd.wait()
        plsc.subcore_barrier()
        sd0.wait()
        sd1.wait()

        gd = [None] * num_chunks
        sd = [None] * num_chunks
        for i in range(num_chunks):
            b = i % NBUF
            if i >= NBUF:
                sd[i - NBUF].wait()
            gd[i] = pltpu.async_copy(z_hbm.at[src_all.at[i]],
                                     rows[b], gsems[b])
            j = i - LAG
            if j >= 0:
                gd[j].wait()
                sd[j] = pltpu.async_copy(rows[j % NBUF],
                                         acc.at[dst_all.at[j]],
                                         ssems[j % NBUF], add=True)
        for j in range(num_chunks - LAG, num_chunks):
            gd[j].wait()
            sd[j] = pltpu.async_copy(rows[j % NBUF],
                                     acc.at[dst_all.at[j]],
                                     ssems[j % NBUF], add=True)
        for j in range(max(0, num_chunks - NBUF), num_chunks):
            sd[j].wait()
        plsc.subcore_barrier()

        pltpu.sync_copy(acc.at[pl.ds(sid * RPT, RPT)],
                        out_hbm.at[cid, pl.ds(sid * RPT, RPT)])

    return edge_pass


def _make_deg_pass(num_chunks: int):
    mesh = plsc.VectorSubcoreMesh(core_axis_name="c", subcore_axis_name="s")

    @functools.partial(
        pl.kernel,
        out_type=jax.ShapeDtypeStruct((NC, NPAD, 16), jnp.float32),
        mesh=mesh,
        compiler_params=pltpu.CompilerParams(use_tc_tiling_on_sc=False),
        scratch_types=[
            pltpu.VMEM((num_chunks, C), jnp.int32),
            pltpu.VMEM((C, 16), jnp.float32),
            pltpu.VMEM_SHARED((NPAD, 16), jnp.float32),
            [pltpu.SemaphoreType.DMA] * 3,
            [pltpu.SemaphoreType.DMA] * NBUF,
        ],
    )
    def deg_pass(ones_hbm, dst_hbm, zeros_hbm, out_hbm,
                 dst_all, ones, acc, sems, ssems):
        cid = lax.axis_index("c")
        sid = lax.axis_index("s")
        wid = sid * NC + cid

        zd = pltpu.async_copy(zeros_hbm,
                              acc.at[pl.ds(sid * RPT, RPT)], sems[0])
        od = pltpu.async_copy(ones_hbm, ones, sems[1])
        dd = pltpu.async_copy(dst_hbm.at[wid], dst_all, sems[2])
        zd.wait()
        plsc.subcore_barrier()
        od.wait()
        dd.wait()

        sd = [None] * num_chunks
        for i in range(num_chunks):
            if i >= NBUF:
                sd[i - NBUF].wait()
            sd[i] = pltpu.async_copy(ones, acc.at[dst_all.at[i]],
                                     ssems[i % NBUF], add=True)
        for j in range(max(0, num_chunks - NBUF), num_chunks):
            sd[j].wait()
        plsc.subcore_barrier()

        pltpu.sync_copy(acc.at[pl.ds(sid * (NPAD // NS), NPAD // NS)],
                        out_hbm.at[cid, pl.ds(sid * (NPAD // NS), NPAD // NS)])

    return deg_pass


def _mlp(x_pad, deg_accs, Ws, bs):
    R = 512
    G = NPAD // R

    def body(x_ref, a_ref, *refs):
        w_refs = refs[:KL]
        b_refs = refs[KL:2 * KL]
        x0_ref, d_ref, z_ref = refs[2 * KL:2 * KL + 3]
        h = x_ref[...]
        for i in range(KL):
            h = jnp.dot(h, w_refs[i][...], preferred_element_type=jnp.float32)
            h = h + b_refs[i][...]
            if i != KL - 1:
                h = jnp.maximum(h, 0.0)
        x0_ref[...] = h
        g = pl.program_id(0)
        deg = a_ref[0, :, 0:1] + a_ref[1, :, 0:1]
        row = g * R + lax.broadcasted_iota(jnp.int32, (R, 1), 0)
        d = jnp.where(row < N, lax.rsqrt(jnp.maximum(deg, 1.0)), 0.0)
        d64 = jnp.broadcast_to(d, (R, NCLS))
        d_ref[...] = d64
        z_ref[...] = d64 * h

    in_specs = [pl.BlockSpec((R, D_IN), lambda i: (i, 0)),
                pl.BlockSpec((NC, R, 16), lambda i: (0, i, 0))]
    for W in Ws:
        in_specs.append(pl.BlockSpec(W.shape, lambda i: (0, 0)))
    for b in bs:
        in_specs.append(pl.BlockSpec((1, b.shape[0]), lambda i: (0, 0)))
    return pl.pallas_call(
        body, grid=(G,), in_specs=in_specs,
        out_specs=[pl.BlockSpec((R, NCLS), lambda i: (i, 0))] * 3,
        out_shape=[jax.ShapeDtypeStruct((NPAD, NCLS), jnp.float32)] * 3,
    )(x_pad, deg_accs, *Ws, *[b[None, :] for b in bs])


def _combine(accs, dinv64, z0):
    NR = NPAD * NCLS // 128
    R = 512
    a = accs.reshape(NC, NR, 128)
    d = dinv64.reshape(NR, 128)
    z = z0.reshape(NR, 128)

    def body(a_ref, d_ref, z_ref, o_ref):
        dd = d_ref[...]
        o_ref[...] = ((1.0 - ALPHA) * dd * dd * (a_ref[0] + a_ref[1])
                      + ALPHA * z_ref[...])

    out = pl.pallas_call(
        body, grid=(NR // R,),
        in_specs=[pl.BlockSpec((NC, R, 128), lambda i: (0, i, 0)),
                  pl.BlockSpec((R, 128), lambda i: (i, 0)),
                  pl.BlockSpec((R, 128), lambda i: (i, 0))],
        out_specs=pl.BlockSpec((R, 128), lambda i: (i, 0)),
        out_shape=jax.ShapeDtypeStruct((NR, 128), jnp.float32),
    )(a, d, z)
    return out.reshape(NPAD, NCLS)


def _final(accs, dinv64, x0):
    R = 512

    def body(a_ref, d_ref, x_ref, o_ref):
        h = ((1.0 - ALPHA) * d_ref[...] * (a_ref[0] + a_ref[1])
             + ALPHA * x_ref[...])
        m = jnp.max(h, axis=1, keepdims=True)
        e = jnp.exp(h - m)
        s = jnp.sum(e, axis=1, keepdims=True)
        o_ref[...] = h - m - jnp.log(s)

    return pl.pallas_call(
        body, grid=(NPAD // R,),
        in_specs=[pl.BlockSpec((NC, R, NCLS), lambda i: (0, i, 0)),
                  pl.BlockSpec((R, NCLS), lambda i: (i, 0)),
                  pl.BlockSpec((R, NCLS), lambda i: (i, 0))],
        out_specs=pl.BlockSpec((R, NCLS), lambda i: (i, 0)),
        out_shape=jax.ShapeDtypeStruct((NPAD, NCLS), jnp.float32),
    )(accs, dinv64, x0)


def kernel(x, edge_index, Ws, bs):
    E = edge_index.shape[1]
    etot = E + N
    num_chunks = -(-etot // (NW * C))
    epad = NW * num_chunks * C

    src = edge_index[0].astype(jnp.int32)
    dst = edge_index[1].astype(jnp.int32)
    loop = jnp.arange(N, dtype=jnp.int32)
    fill = N + jnp.arange(epad - etot, dtype=jnp.int32) % (NPAD - N)
    src3 = jnp.concatenate([src, loop, fill]).reshape(NW, num_chunks, C)
    dst3 = jnp.concatenate([dst, loop, fill]).reshape(NW, num_chunks, C)
    zeros_chunk = jnp.zeros((RPT, NCLS), jnp.float32)

    deg_pass = _make_deg_pass(num_chunks)
    deg_accs = deg_pass(jnp.ones((C, 16), jnp.float32), dst3,
                        jnp.zeros((RPT, 16), jnp.float32))
    x_pad = jnp.zeros((NPAD, D_IN), jnp.float32).at[:N].set(x)
    x0, dinv64, z0 = _mlp(x_pad, deg_accs, Ws, bs)

    edge_pass = _make_edge_pass(num_chunks)
    z = z0
    for _ in range(KL - 1):
        accs = edge_pass(z, src3, dst3, zeros_chunk)
        z = _combine(accs, dinv64, z0)
    accs = edge_pass(z, src3, dst3, zeros_chunk)
    out_pad = _final(accs, dinv64, x0)
    return out_pad[:N]


# speedup vs baseline: 1.0302x; 1.0302x over previous
import functools

import jax
import jax.numpy as jnp
from jax import lax
from jax.experimental import pallas as pl
from jax.experimental.pallas import tpu as pltpu
from jax.experimental.pallas import tpu_sc as plsc

N = 10000
D_IN = 128
NCLS = 64
KL = 10
ALPHA = 0.1

NC = 2
NS = 16
NW = NC * NS
C = 128
NPAD = 10240
RPT = NPAD // NS

NBUF = 7
LAG = 3


def _make_edge_pass(num_chunks: int):
    assert num_chunks > NBUF
    mesh = plsc.VectorSubcoreMesh(core_axis_name="c", subcore_axis_name="s")

    @functools.partial(
        pl.kernel,
        out_type=jax.ShapeDtypeStruct((NC, NPAD, NCLS), jnp.float32),
        mesh=mesh,
        compiler_params=pltpu.CompilerParams(use_tc_tiling_on_sc=False),
        scratch_types=[
            pltpu.VMEM((num_chunks, C), jnp.int32),
            pltpu.VMEM((num_chunks, C), jnp.int32),
            [pltpu.VMEM((C, NCLS), jnp.float32)] * NBUF,
            pltpu.VMEM_SHARED((NPAD, NCLS), jnp.float32),
            [pltpu.SemaphoreType.DMA] * 3,
            [pltpu.SemaphoreType.DMA] * NBUF,
            [pltpu.SemaphoreType.DMA] * NBUF,
        ],
    )
    def edge_pass(z_hbm, src_hbm, dst_hbm, zeros_hbm, out_hbm,
                  src_all, dst_all, rows, acc, sems, gsems, ssems):
        cid = lax.axis_index("c")
        sid = lax.axis_index("s")
        wid = sid * NC + cid

        zd = pltpu.async_copy(zeros_hbm, acc.at[pl.ds(sid * RPT, RPT)],
                              sems[0])
        sd0 = pltpu.async_copy(src_hbm.at[wid], src_all, sems[1])
        sd1 = pltpu.async_copy(dst_hbm.at[wid], dst_all, sems[2])
        zd.wait()
        plsc.subcore_barrier()
        sd0.wait()
        sd1.wait()

        gd = [None] * num_chunks
        sd = [None] * num_chunks
        for i in range(num_chunks):
            b = i % NBUF
            if i >= NBUF:
                sd[i - NBUF].wait()
            gd[i] = pltpu.async_copy(z_hbm.at[src_all.at[i]],
                                     rows[b], gsems[b])
            j = i - LAG
            if j >= 0:
                gd[j].wait()
                sd[j] = pltpu.async_copy(rows[j % NBUF],
                                         acc.at[dst_all.at[j]],
                                         ssems[j % NBUF], add=True)
        for j in range(num_chunks - LAG, num_chunks):
            gd[j].wait()
            sd[j] = pltpu.async_copy(rows[j % NBUF],
                                     acc.at[dst_all.at[j]],
                                     ssems[j % NBUF], add=True)
        for j in range(max(0, num_chunks - NBUF), num_chunks):
            sd[j].wait()
        plsc.subcore_barrier()

        pltpu.sync_copy(acc.at[pl.ds(sid * RPT, RPT)],
                        out_hbm.at[cid, pl.ds(sid * RPT, RPT)])

    return edge_pass


def _make_deg_pass(num_chunks: int):
    mesh = plsc.VectorSubcoreMesh(core_axis_name="c", subcore_axis_name="s")

    @functools.partial(
        pl.kernel,
        out_type=jax.ShapeDtypeStruct((NC, NPAD, 16), jnp.float32),
        mesh=mesh,
        compiler_params=pltpu.CompilerParams(use_tc_tiling_on_sc=False),
        scratch_types=[
            pltpu.VMEM((num_chunks, C), jnp.int32),
            pltpu.VMEM((C, 16), jnp.float32),
            pltpu.VMEM_SHARED((NPAD, 16), jnp.float32),
            [pltpu.SemaphoreType.DMA] * 3,
            [pltpu.SemaphoreType.DMA] * NBUF,
        ],
    )
    def deg_pass(ones_hbm, dst_hbm, zeros_hbm, out_hbm,
                 dst_all, ones, acc, sems, ssems):
        cid = lax.axis_index("c")
        sid = lax.axis_index("s")
        wid = sid * NC + cid

        zd = pltpu.async_copy(zeros_hbm,
                              acc.at[pl.ds(sid * RPT, RPT)], sems[0])
        od = pltpu.async_copy(ones_hbm, ones, sems[1])
        dd = pltpu.async_copy(dst_hbm.at[wid], dst_all, sems[2])
        zd.wait()
        plsc.subcore_barrier()
        od.wait()
        dd.wait()

        sd = [None] * num_chunks
        for i in range(num_chunks):
            if i >= NBUF:
                sd[i - NBUF].wait()
            sd[i] = pltpu.async_copy(ones, acc.at[dst_all.at[i]],
                                     ssems[i % NBUF], add=True)
        for j in range(max(0, num_chunks - NBUF), num_chunks):
            sd[j].wait()
        plsc.subcore_barrier()

        pltpu.sync_copy(acc.at[pl.ds(sid * (NPAD // NS), NPAD // NS)],
                        out_hbm.at[cid, pl.ds(sid * (NPAD // NS), NPAD // NS)])

    return deg_pass


def _mlp(x_pad, deg_accs, Ws, bs):
    R = 512
    G = NPAD // R

    def body(x_ref, a_ref, *refs):
        w_refs = refs[:KL]
        b_refs = refs[KL:2 * KL]
        x0_ref, d_ref, z_ref = refs[2 * KL:2 * KL + 3]
        h = x_ref[...]
        for i in range(KL):
            h = jnp.dot(h, w_refs[i][...], preferred_element_type=jnp.float32)
            h = h + b_refs[i][...]
            if i != KL - 1:
                h = jnp.maximum(h, 0.0)
        x0_ref[...] = h
        g = pl.program_id(0)
        deg = a_ref[0, :, 0:1] + a_ref[1, :, 0:1]
        row = g * R + lax.broadcasted_iota(jnp.int32, (R, 1), 0)
        d = jnp.where(row < N, lax.rsqrt(jnp.maximum(deg, 1.0)), 0.0)
        d64 = jnp.broadcast_to(d, (R, NCLS))
        d_ref[...] = d64
        z_ref[...] = d64 * h

    in_specs = [pl.BlockSpec((R, D_IN), lambda i: (i, 0)),
                pl.BlockSpec((NC, R, 16), lambda i: (0, i, 0))]
    for W in Ws:
        in_specs.append(pl.BlockSpec(W.shape, lambda i: (0, 0)))
    for b in bs:
        in_specs.append(pl.BlockSpec((1, b.shape[0]), lambda i: (0, 0)))
    return pl.pallas_call(
        body, grid=(G,), in_specs=in_specs,
        out_specs=[pl.BlockSpec((R, NCLS), lambda i: (i, 0))] * 3,
        out_shape=[jax.ShapeDtypeStruct((NPAD, NCLS), jnp.float32)] * 3,
    )(x_pad, deg_accs, *Ws, *[b[None, :] for b in bs])


def _combine(accs, dinv64, z0):
    NR = NPAD * NCLS // 128
    R = 512
    a = accs.reshape(NC, NR, 128)
    d = dinv64.reshape(NR, 128)
    z = z0.reshape(NR, 128)

    def body(a_ref, d_ref, z_ref, o_ref):
        dd = d_ref[...]
        o_ref[...] = ((1.0 - ALPHA) * dd * dd * (a_ref[0] + a_ref[1])
                      + ALPHA * z_ref[...])

    out = pl.pallas_call(
        body, grid=(NR // R,),
        in_specs=[pl.BlockSpec((NC, R, 128), lambda i: (0, i, 0)),
                  pl.BlockSpec((R, 128), lambda i: (i, 0)),
                  pl.BlockSpec((R, 128), lambda i: (i, 0))],
        out_specs=pl.BlockSpec((R, 128), lambda i: (i, 0)),
        out_shape=jax.ShapeDtypeStruct((NR, 128), jnp.float32),
    )(a, d, z)
    return out.reshape(NPAD, NCLS)


def _final(accs, dinv64, x0):
    R = 512

    def body(a_ref, d_ref, x_ref, o_ref):
        h = ((1.0 - ALPHA) * d_ref[...] * (a_ref[0] + a_ref[1])
             + ALPHA * x_ref[...])
        m = jnp.max(h, axis=1, keepdims=True)
        e = jnp.exp(h - m)
        s = jnp.sum(e, axis=1, keepdims=True)
        o_ref[...] = h - m - jnp.log(s)

    return pl.pallas_call(
        body, grid=(NPAD // R,),
        in_specs=[pl.BlockSpec((NC, R, NCLS), lambda i: (0, i, 0)),
                  pl.BlockSpec((R, NCLS), lambda i: (i, 0)),
                  pl.BlockSpec((R, NCLS), lambda i: (i, 0))],
        out_specs=pl.BlockSpec((R, NCLS), lambda i: (i, 0)),
        out_shape=jax.ShapeDtypeStruct((NPAD, NCLS), jnp.float32),
    )(accs, dinv64, x0)


def kernel(x, edge_index, Ws, bs):
    E = edge_index.shape[1]
    etot = E + N
    num_chunks = -(-etot // (NW * C))
    epad = NW * num_chunks * C

    src = edge_index[0].astype(jnp.int32)
    dst = edge_index[1].astype(jnp.int32)
    loop = jnp.arange(N, dtype=jnp.int32)
    fill = N + jnp.arange(epad - etot, dtype=jnp.int32) % (NPAD - N)
    src3 = jnp.concatenate([src, loop, fill]).reshape(NW, num_chunks, C)
    dst3 = jnp.concatenate([dst, loop, fill]).reshape(NW, num_chunks, C)
    zeros_chunk = jnp.zeros((RPT, NCLS), jnp.float32)

    deg_pass = _make_deg_pass(num_chunks)
    deg_accs = deg_pass(jnp.ones((C, 16), jnp.float32), dst3,
                        jnp.zeros((RPT, 16), jnp.float32))
    x_pad = jnp.zeros((NPAD, D_IN), jnp.float32).at[:N].set(x)
    x0, dinv64, z0 = _mlp(x_pad, deg_accs, Ws, bs)

    edge_pass = _make_edge_pass(num_chunks)
    z = z0
    for _ in range(KL - 1):
        accs = edge_pass(z, src3, dst3, zeros_chunk)
        z = _combine(accs, dinv64, z0)
    accs = edge_pass(z, src3, dst3, zeros_chunk)
    out_pad = _final(accs, dinv64, x0)
    return out_pad[:N]


# speedup vs baseline: 1.0936x; 1.0616x over previous
import functools

import jax
import jax.numpy as jnp
from jax import lax
from jax.experimental import pallas as pl
from jax.experimental.pallas import tpu as pltpu
from jax.experimental.pallas import tpu_sc as plsc

N = 10000
D_IN = 128
NCLS = 64
KL = 10
ALPHA = 0.1

NC = 2
NS = 16
NW = NC * NS
C = 128
NPAD = 10240
RPT = NPAD // NS

NBUF = 6
LAG = 3


def _make_edge_pass(num_chunks: int):
    assert num_chunks > NBUF
    mesh = plsc.VectorSubcoreMesh(core_axis_name="c", subcore_axis_name="s")

    @functools.partial(
        pl.kernel,
        out_type=jax.ShapeDtypeStruct((NC, NPAD, NCLS), jnp.float32),
        mesh=mesh,
        compiler_params=pltpu.CompilerParams(use_tc_tiling_on_sc=False),
        scratch_types=[
            pltpu.VMEM((num_chunks, C), jnp.int32),
            pltpu.VMEM((num_chunks, C), jnp.int32),
            [pltpu.VMEM((C, NCLS), jnp.float32)] * NBUF,
            pltpu.VMEM_SHARED((NPAD, NCLS), jnp.float32),
            [pltpu.SemaphoreType.DMA] * 3,
            [pltpu.SemaphoreType.DMA] * NBUF,
            [pltpu.SemaphoreType.DMA] * NBUF,
        ],
    )
    def edge_pass(z_hbm, src_hbm, dst_hbm, zeros_hbm, out_hbm,
                  src_all, dst_all, rows, acc, sems, gsems, ssems):
        cid = lax.axis_index("c")
        sid = lax.axis_index("s")
        wid = sid * NC + cid

        zd = pltpu.async_copy(zeros_hbm, acc.at[pl.ds(sid * RPT, RPT)],
                              sems[0])
        sd0 = pltpu.async_copy(src_hbm.at[wid], src_all, sems[1])
        sd1 = pltpu.async_copy(dst_hbm.at[wid], dst_all, sems[2])
        zd.wait()
        plsc.subcore_barrier()
        sd0.wait()
        sd1.wait()

        gd = [None] * num_chunks
        sd = [None] * num_chunks
        for i in range(num_chunks):
            b = i % NBUF
            if i >= NBUF:
                sd[i - NBUF].wait()
            gd[i] = pltpu.async_copy(z_hbm.at[src_all.at[i]],
                                     rows[b], gsems[b])
            j = i - LAG
            if j >= 0:
                gd[j].wait()
                sd[j] = pltpu.async_copy(rows[j % NBUF],
                                         acc.at[dst_all.at[j]],
                                         ssems[j % NBUF], add=True)
        for j in range(num_chunks - LAG, num_chunks):
            gd[j].wait()
            sd[j] = pltpu.async_copy(rows[j % NBUF],
                                     acc.at[dst_all.at[j]],
                                     ssems[j % NBUF], add=True)
        for j in range(max(0, num_chunks - NBUF), num_chunks):
            sd[j].wait()
        plsc.subcore_barrier()

        pltpu.sync_copy(acc.at[pl.ds(sid * RPT, RPT)],
                        out_hbm.at[cid, pl.ds(sid * RPT, RPT)])

    return edge_pass


def _make_deg_pass(num_chunks: int):
    mesh = plsc.VectorSubcoreMesh(core_axis_name="c", subcore_axis_name="s")

    @functools.partial(
        pl.kernel,
        out_type=jax.ShapeDtypeStruct((NC, NPAD, 16), jnp.float32),
        mesh=mesh,
        compiler_params=pltpu.CompilerParams(use_tc_tiling_on_sc=False),
        scratch_types=[
            pltpu.VMEM((num_chunks, C), jnp.int32),
            pltpu.VMEM((C, 16), jnp.float32),
            pltpu.VMEM_SHARED((NPAD, 16), jnp.float32),
            [pltpu.SemaphoreType.DMA] * 3,
            [pltpu.SemaphoreType.DMA] * NBUF,
        ],
    )
    def deg_pass(ones_hbm, dst_hbm, zeros_hbm, out_hbm,
                 dst_all, ones, acc, sems, ssems):
        cid = lax.axis_index("c")
        sid = lax.axis_index("s")
        wid = sid * NC + cid

        zd = pltpu.async_copy(zeros_hbm,
                              acc.at[pl.ds(sid * RPT, RPT)], sems[0])
        od = pltpu.async_copy(ones_hbm, ones, sems[1])
        dd = pltpu.async_copy(dst_hbm.at[wid], dst_all, sems[2])
        zd.wait()
        plsc.subcore_barrier()
        od.wait()
        dd.wait()

        sd = [None] * num_chunks
        for i in range(num_chunks):
            if i >= NBUF:
                sd[i - NBUF].wait()
            sd[i] = pltpu.async_copy(ones, acc.at[dst_all.at[i]],
                                     ssems[i % NBUF], add=True)
        for j in range(max(0, num_chunks - NBUF), num_chunks):
            sd[j].wait()
        plsc.subcore_barrier()

        pltpu.sync_copy(acc.at[pl.ds(sid * (NPAD // NS), NPAD // NS)],
                        out_hbm.at[cid, pl.ds(sid * (NPAD // NS), NPAD // NS)])

    return deg_pass


def _mlp(x_pad, deg_accs, Ws, bs):
    R = 512
    G = NPAD // R

    def body(x_ref, a_ref, *refs):
        w_refs = refs[:KL]
        b_refs = refs[KL:2 * KL]
        x0_ref, d_ref, z_ref = refs[2 * KL:2 * KL + 3]
        h = x_ref[...]
        for i in range(KL):
            h = jnp.dot(h, w_refs[i][...], preferred_element_type=jnp.float32)
            h = h + b_refs[i][...]
            if i != KL - 1:
                h = jnp.maximum(h, 0.0)
        x0_ref[...] = h
        g = pl.program_id(0)
        deg = a_ref[0, :, 0:1] + a_ref[1, :, 0:1]
        row = g * R + lax.broadcasted_iota(jnp.int32, (R, 1), 0)
        d = jnp.where(row < N, lax.rsqrt(jnp.maximum(deg, 1.0)), 0.0)
        d64 = jnp.broadcast_to(d, (R, NCLS))
        d_ref[...] = d64
        z_ref[...] = d64 * h

    in_specs = [pl.BlockSpec((R, D_IN), lambda i: (i, 0)),
                pl.BlockSpec((NC, R, 16), lambda i: (0, i, 0))]
    for W in Ws:
        in_specs.append(pl.BlockSpec(W.shape, lambda i: (0, 0)))
    for b in bs:
        in_specs.append(pl.BlockSpec((1, b.shape[0]), lambda i: (0, 0)))
    return pl.pallas_call(
        body, grid=(G,), in_specs=in_specs,
        out_specs=[pl.BlockSpec((R, NCLS), lambda i: (i, 0))] * 3,
        out_shape=[jax.ShapeDtypeStruct((NPAD, NCLS), jnp.float32)] * 3,
    )(x_pad, deg_accs, *Ws, *[b[None, :] for b in bs])


def _combine(accs, dinv64, z0):
    NR = NPAD * NCLS // 128
    R = 512
    a = accs.reshape(NC, NR, 128)
    d = dinv64.reshape(NR, 128)
    z = z0.reshape(NR, 128)

    def body(a_ref, d_ref, z_ref, o_ref):
        dd = d_ref[...]
        o_ref[...] = ((1.0 - ALPHA) * dd * dd * (a_ref[0] + a_ref[1])
                      + ALPHA * z_ref[...])

    out = pl.pallas_call(
        body, grid=(NR // R,),
        in_specs=[pl.BlockSpec((NC, R, 128), lambda i: (0, i, 0)),
                  pl.BlockSpec((R, 128), lambda i: (i, 0)),
                  pl.BlockSpec((R, 128), lambda i: (i, 0))],
        out_specs=pl.BlockSpec((R, 128), lambda i: (i, 0)),
        out_shape=jax.ShapeDtypeStruct((NR, 128), jnp.float32),
    )(a, d, z)
    return out.reshape(NPAD, NCLS)


def _final(accs, dinv64, x0):
    R = 512

    def body(a_ref, d_ref, x_ref, o_ref):
        h = ((1.0 - ALPHA) * d_ref[...] * (a_ref[0] + a_ref[1])
             + ALPHA * x_ref[...])
        m = jnp.max(h, axis=1, keepdims=True)
        e = jnp.exp(h - m)
        s = jnp.sum(e, axis=1, keepdims=True)
        o_ref[...] = h - m - jnp.log(s)

    return pl.pallas_call(
        body, grid=(NPAD // R,),
        in_specs=[pl.BlockSpec((NC, R, NCLS), lambda i: (0, i, 0)),
                  pl.BlockSpec((R, NCLS), lambda i: (i, 0)),
                  pl.BlockSpec((R, NCLS), lambda i: (i, 0))],
        out_specs=pl.BlockSpec((R, NCLS), lambda i: (i, 0)),
        out_shape=jax.ShapeDtypeStruct((NPAD, NCLS), jnp.float32),
    )(accs, dinv64, x0)


def kernel(x, edge_index, Ws, bs):
    E = edge_index.shape[1]
    etot = E + N
    num_chunks = -(-etot // (NW * C))
    epad = NW * num_chunks * C

    src = edge_index[0].astype(jnp.int32)
    dst = edge_index[1].astype(jnp.int32)
    loop = jnp.arange(N, dtype=jnp.int32)
    fill = N + jnp.arange(epad - etot, dtype=jnp.int32) % (NPAD - N)
    src3 = jnp.concatenate([src, loop, fill]).reshape(NW, num_chunks, C)
    dst3 = jnp.concatenate([dst, loop, fill]).reshape(NW, num_chunks, C)
    zeros_chunk = jnp.zeros((RPT, NCLS), jnp.float32)

    deg_pass = _make_deg_pass(num_chunks)
    deg_accs = deg_pass(jnp.ones((C, 16), jnp.float32), dst3,
                        jnp.zeros((RPT, 16), jnp.float32))
    x_pad = jnp.zeros((NPAD, D_IN), jnp.float32).at[:N].set(x)
    x0, dinv64, z0 = _mlp(x_pad, deg_accs, Ws, bs)

    edge_pass = _make_edge_pass(num_chunks)
    z = z0
    for _ in range(KL - 1):
        accs = edge_pass(z, src3, dst3, zeros_chunk)
        z = _combine(accs, dinv64, z0)
    accs = edge_pass(z, src3, dst3, zeros_chunk)
    out_pad = _final(accs, dinv64, x0)
    return out_pad[:N]
